# baseline (speedup 1.0000x reference)
import functools

import jax
import jax.numpy as jnp
from jax import lax
from jax.experimental import pallas as pl
from jax.experimental.pallas import tpu as pltpu
from jax.experimental.pallas import tpu_sc as plsc

BATCH = 4096
MAXLEN = 200
EMBED = 32

_NC = 2
_NS = 16
_NW = _NC * _NS
_B_PER_W = BATCH // _NW
_KB = 4
_CH = _KB * MAXLEN
_NCH = _B_PER_W // _KB
_N_PER_W = _B_PER_W * MAXLEN

_R_SPLITS = [(0, 128), (128, 72)]


def _issue_gather(tok_hbm, idx_all, rows_b, sem, base):
    for q in range(_KB):
        for s, sz in _R_SPLITS:
            pltpu.async_copy(
                tok_hbm.at[idx_all.at[pl.ds(base + q * MAXLEN + s, sz)]],
                rows_b.at[q].at[pl.ds(s, sz)],
                sem,
            )


def _wait_gather(tok_hbm, rows_b, sem):
    for q in range(_KB):
        pltpu.make_async_copy(
            tok_hbm.at[pl.ds(0, MAXLEN)], rows_b.at[q], sem
        ).wait()


def _add_pos(rows_b, pos_v):
    def add_t(t, c):
        p0 = pos_v[t, pl.ds(0, 16)]
        p1 = pos_v[t, pl.ds(16, 16)]
        for q in range(_KB):
            rows_b[q, t, pl.ds(0, 16)] += p0
            rows_b[q, t, pl.ds(16, 16)] += p1
        return c

    lax.fori_loop(0, MAXLEN, add_t, 0, unroll=2)


def _emb_body(x_hbm, tok_hbm, pos_hbm, out_hbm,
              idx_all, pos_v, rows2, gsem0, gsem1, osem0, osem1):
    wid = lax.axis_index("s") * _NC + lax.axis_index("c")
    wbase = wid * _N_PER_W
    brow0 = wid * _B_PER_W

    buf0 = rows2.at[0]
    buf1 = rows2.at[1]

    pltpu.sync_copy(pos_hbm, pos_v)
    pltpu.sync_copy(x_hbm.at[pl.ds(wbase, _N_PER_W)], idx_all)

    _issue_gather(tok_hbm, idx_all, buf0, gsem0, 0)

    def outer(j, carry):
        ca = 2 * j
        cb = 2 * j + 1

        _wait_gather(tok_hbm, buf0, gsem0)

        @pl.when(j > 0)
        def _():
            pltpu.make_async_copy(buf1, out_hbm.at[pl.ds(0, _KB)], osem1).wait()

        _issue_gather(tok_hbm, idx_all, buf1, gsem1, cb * _CH)
        _add_pos(buf0, pos_v)
        pltpu.async_copy(buf0, out_hbm.at[pl.ds(brow0 + ca * _KB, _KB)], osem0)

        _wait_gather(tok_hbm, buf1, gsem1)
        pltpu.make_async_copy(buf0, out_hbm.at[pl.ds(0, _KB)], osem0).wait()

        @pl.when(j < _NCH // 2 - 1)
        def _():
            _issue_gather(tok_hbm, idx_all, buf0, gsem0, (ca + 2) * _CH)

        _add_pos(buf1, pos_v)
        pltpu.async_copy(buf1, out_hbm.at[pl.ds(brow0 + cb * _KB, _KB)], osem1)
        return carry

    lax.fori_loop(0, _NCH // 2, outer, 0)

    pltpu.make_async_copy(buf1, out_hbm.at[pl.ds(0, _KB)], osem1).wait()


@jax.jit
def _emb_call(x_flat, token_emb, pos_emb):
    mesh = plsc.VectorSubcoreMesh(core_axis_name="c", subcore_axis_name="s")
    k = functools.partial(
        pl.kernel,
        mesh=mesh,
        out_type=jax.ShapeDtypeStruct((BATCH, MAXLEN, EMBED), jnp.float32),
        scratch_types=[
            pltpu.VMEM((_N_PER_W,), jnp.int32),
            pltpu.VMEM((MAXLEN, EMBED), jnp.float32),
            pltpu.VMEM((2, _KB, MAXLEN, EMBED), jnp.float32),
            pltpu.SemaphoreType.DMA,
            pltpu.SemaphoreType.DMA,
            pltpu.SemaphoreType.DMA,
            pltpu.SemaphoreType.DMA,
        ],
        compiler_params=pltpu.CompilerParams(use_tc_tiling_on_sc=False),
    )(_emb_body)
    return k(x_flat, token_emb, pos_emb)


def kernel(x, token_emb, pos_emb):
    return _emb_call(x.reshape(-1).astype(jnp.int32), token_emb, pos_emb)
